# baseline (speedup 1.0000x reference)
import jax
import jax.numpy as jnp
from jax import lax
from jax.experimental import pallas as pl
from jax.experimental.pallas import tpu as pltpu
from jax.experimental.pallas import tpu_sc as plsc

NC = 2
NS = 16
L = 16
NW = NC * NS

NUM_ROWS = 1000000
BATCH = 16384
DIM = 32
SUB = 8
NBLK = DIM // SUB
TILES = -(-NUM_ROWS // 128)
TABLE_TILES = NBLK * TILES
KMAX = -(-TABLE_TILES // NW)
BLOCK_WORDS = TILES * 1024

B_PER_W = BATCH // NW
IDXC = 128
N_IDXC = B_PER_W // IDXC
GROUPS = B_PER_W // L
NBUF = 32


def _p1_body(umat_hbm, imat_hbm, uout_hbm, iout_hbm, tv_v, rsem, wsem):
  wid = lax.axis_index("s") * NC + lax.axis_index("c")

  for src, dst in ((umat_hbm, uout_hbm), (imat_hbm, iout_hbm)):
    def tile_of(k):
      t = (wid + k * NW) % TABLE_TILES
      return t // TILES, t % TILES

    def start_read(k, slot):
      b, c = tile_of(k)
      pltpu.async_copy(src.at[b, slice(None), pl.ds(c * 128, 128)],
                       tv_v.at[slot], rsem)

    def wait_one(sem):
      pltpu.make_async_copy(dst.at[0], tv_v.at[0], sem).wait()

    for kk in range(NBUF - 1):
      start_read(kk, kk)

    def step(k, carry):
      start_read((k + NBUF - 1) % KMAX, (k + NBUF - 1) % NBUF)
      wait_one(rsem)
      b, c = tile_of(k)
      pltpu.async_copy(tv_v.at[k % NBUF], dst.at[b * TILES + c], wsem)

      @pl.when(k >= NBUF - 1)
      def _drain_one():
        wait_one(wsem)
      return carry

    lax.fori_loop(0, KMAX, step, 0, unroll=False)
    for _ in range(NBUF - 1):
      wait_one(rsem)
      wait_one(wsem)


def _p2_body(uid_hbm, cid_hbm, uflat_hbm, iflat_hbm, out_hbm,
             uids_v, cids_v, uidx_v, cidx_v, ubuf_v, ibuf_v, out_v, sem):
  wid = lax.axis_index("s") * NC + lax.axis_index("c")
  base = wid * B_PER_W

  for j in range(N_IDXC):
    pltpu.sync_copy(uid_hbm.at[pl.ds(base + j * IDXC, IDXC)], uids_v.at[j])
    pltpu.sync_copy(cid_hbm.at[pl.ds(base + j * IDXC, IDXC)], cids_v.at[j])

  def build(j, carry):
    for k in range(IDXC // L):
      s = pl.ds(k * L, L)
      for ids_v, idx_v in ((uids_v, uidx_v), (cids_v, cidx_v)):
        idv = ids_v[j, s]
        bvec = lax.shift_left(lax.shift_right_logical(idv, 7), 10) + \
            (idv & jnp.full((L,), 127, jnp.int32))
        for d in range(DIM):
          off = (d // SUB) * BLOCK_WORDS + (d % SUB) * 128
          idx_v[j, d, s] = bvec + jnp.full((L,), off, jnp.int32)
    return carry

  lax.fori_loop(0, N_IDXC, build, 0, unroll=False)

  copies = []
  for j in range(N_IDXC):
    s = pl.ds(j * IDXC, IDXC)
    for d in range(DIM):
      copies.append(pltpu.async_copy(
          uflat_hbm.at[uidx_v.at[j, d]], ubuf_v.at[d, s], sem))
      copies.append(pltpu.async_copy(
          iflat_hbm.at[cidx_v.at[j, d]], ibuf_v.at[d, s], sem))
  for c in copies:
    c.wait()

  def compute(g, carry):
    s = pl.ds(pl.multiple_of(g * L, L), L)
    acc = jnp.zeros((L,), jnp.float32)
    for d in range(DIM):
      acc = acc + ubuf_v[d, s] * ibuf_v[d, s]
    e = jnp.exp(-jnp.abs(acc))
    sig = jnp.where(acc >= 0, 1.0 / (1.0 + e), e / (1.0 + e))
    out_v[s] = sig
    return carry

  lax.fori_loop(0, GROUPS, compute, 0, unroll=False)

  pltpu.sync_copy(out_v, out_hbm.at[pl.ds(base, B_PER_W)])


@jax.jit
def kernel(user_ids, content_ids, user_matrix, item_matrix):
  uid = user_ids.astype(jnp.int32)
  cid = content_ids.astype(jnp.int32)
  umat = user_matrix.T.reshape(NBLK, SUB, NUM_ROWS)
  imat = item_matrix.T.reshape(NBLK, SUB, NUM_ROWS)

  mesh = plsc.VectorSubcoreMesh(
      core_axis_name="c", subcore_axis_name="s", num_cores=NC,
      num_subcores=NS)

  p1 = pl.kernel(
      _p1_body,
      out_type=(jax.ShapeDtypeStruct((TABLE_TILES, SUB, 128), jnp.float32),
                jax.ShapeDtypeStruct((TABLE_TILES, SUB, 128), jnp.float32)),
      mesh=mesh,
      compiler_params=pltpu.CompilerParams(
          needs_layout_passes=False, use_tc_tiling_on_sc=True,
          disable_bounds_checks=True),
      scratch_types=[
          pltpu.VMEM((NBUF, SUB, 128), jnp.float32),
          pltpu.SemaphoreType.DMA,
          pltpu.SemaphoreType.DMA,
      ],
  )
  ubm, ibm = p1(umat, imat)

  p2 = pl.kernel(
      _p2_body,
      out_type=jax.ShapeDtypeStruct((BATCH,), jnp.float32),
      mesh=mesh,
      compiler_params=pltpu.CompilerParams(
          needs_layout_passes=False, use_tc_tiling_on_sc=False,
          disable_bounds_checks=True),
      scratch_types=[
          pltpu.VMEM((N_IDXC, IDXC), jnp.int32),
          pltpu.VMEM((N_IDXC, IDXC), jnp.int32),
          pltpu.VMEM((N_IDXC, DIM, IDXC), jnp.int32),
          pltpu.VMEM((N_IDXC, DIM, IDXC), jnp.int32),
          pltpu.VMEM((DIM, B_PER_W), jnp.float32),
          pltpu.VMEM((DIM, B_PER_W), jnp.float32),
          pltpu.VMEM((B_PER_W,), jnp.float32),
          pltpu.SemaphoreType.DMA,
      ],
  )
  return p2(uid, cid, ubm.reshape(-1), ibm.reshape(-1))
